# Initial kernel scaffold; baseline (speedup 1.0000x reference)
import jax
import jax.numpy as jnp
from jax.experimental import pallas as pl


def kernel(x, edge_index, W_msg0, W_self0, b0, W_msg1, W_self1, b1, fc1_w, fc1_b, fc2_w, fc2_b):
    raise NotImplementedError("write your pallas kernel here")


# speedup vs baseline: 6.3682x; 6.3682x over previous
import functools

import jax
import jax.numpy as jnp
from jax import lax
from jax.experimental import pallas as pl
from jax.experimental.pallas import tpu as pltpu
from jax.experimental.pallas import tpu_sc as plsc

N = 10000
E = 320000
D = 128
DA = 144

NC = 2
NS = 16
NW = NC * NS
EW = E // NW
K = 80
CH = EW // K
NP = 10240
RPT = NP // NS
RST = 32


def _make_sc_segsum(width):
  mesh = plsc.VectorSubcoreMesh(core_axis_name="c", subcore_axis_name="s")

  @functools.partial(
      pl.kernel,
      out_type=jax.ShapeDtypeStruct((2 * NP, width), jnp.float32),
      mesh=mesh,
      compiler_params=pltpu.CompilerParams(use_tc_tiling_on_sc=False),
      scratch_types=[
          pltpu.VMEM((CH, K), jnp.int32),
          pltpu.VMEM((CH, K), jnp.int32),
          pltpu.VMEM((K, width), jnp.float32),
          pltpu.VMEM((RST, width), jnp.float32),
          pltpu.VMEM_SHARED((NP, width), jnp.float32),
          pltpu.SemaphoreType.DMA,
      ],
  )
  def segsum(tab_hbm, src_hbm, dst_hbm, out_hbm,
             src_v, dst_v, rows_v, stage_v, acc_sh, sem):
    cid = lax.axis_index("c")
    sid = lax.axis_index("s")
    wid = cid * NS + sid

    zero = jnp.zeros((16,), jnp.float32)

    def zrow(r, carry):
      def zcol(c, carry2):
        stage_v[r, pl.ds(c * 16, 16)] = zero
        return carry2
      return lax.fori_loop(0, width // 16, zcol, carry)

    lax.fori_loop(0, RST, zrow, 0)
    for j in range(RPT // RST):
      pltpu.sync_copy(stage_v, acc_sh.at[pl.ds(sid * RPT + j * RST, RST)])

    pltpu.sync_copy(src_hbm.at[wid], src_v)
    pltpu.sync_copy(dst_hbm.at[wid], dst_v)

    plsc.subcore_barrier()

    def body(j, carry):
      pltpu.async_copy(tab_hbm.at[src_v.at[j]], rows_v, sem).wait()
      pltpu.sync_copy(rows_v, acc_sh.at[dst_v.at[j]], add=True)
      return carry

    lax.fori_loop(0, CH, body, 0)

    plsc.subcore_barrier()

    for j in range(RPT // RST):
      row = sid * RPT + j * RST
      pltpu.sync_copy(acc_sh.at[pl.ds(row, RST)], stage_v)
      pltpu.sync_copy(stage_v, out_hbm.at[pl.ds(cid * NP + row, RST)])

  return segsum


_sc_segsum_aug = _make_sc_segsum(DA)
_sc_segsum = _make_sc_segsum(D)


_RB = 2000


def _dense_body(p_ref, d_ref, x_ref, wm_ref, ws_ref, b_ref, o_ref):
  a = p_ref[0] + p_ref[1]
  deg = d_ref[:, 0:1] + d_ref[:, 1:2]
  scale = 1.0 / jnp.maximum(deg, 1.0)
  agg = jnp.dot(a * scale, wm_ref[...], preferred_element_type=jnp.float32)
  self_t = jnp.dot(x_ref[...], ws_ref[...], preferred_element_type=jnp.float32)
  o_ref[...] = jnp.maximum(agg + self_t + b_ref[...], 0.0)


def _dense_layer(P, degT, x, Wm, Ws, b):
  return pl.pallas_call(
      _dense_body,
      grid=(N // _RB,),
      in_specs=[
          pl.BlockSpec((2, _RB, D), lambda i: (0, i, 0)),
          pl.BlockSpec((_RB, 2), lambda i: (i, 0)),
          pl.BlockSpec((_RB, D), lambda i: (i, 0)),
          pl.BlockSpec((D, D), lambda i: (0, 0)),
          pl.BlockSpec((D, D), lambda i: (0, 0)),
          pl.BlockSpec((1, D), lambda i: (0, 0)),
      ],
      out_specs=pl.BlockSpec((_RB, D), lambda i: (i, 0)),
      out_shape=jax.ShapeDtypeStruct((N, D), jnp.float32),
  )(P, degT, x, Wm, Ws, b.reshape(1, D))


def _head_body(h_ref, w1_ref, b1_ref, w2_ref, b2_ref, o_ref):
  g = jnp.max(h_ref[...], axis=0, keepdims=True)
  h1 = jnp.maximum(
      jnp.dot(g, w1_ref[...], preferred_element_type=jnp.float32) + b1_ref[...],
      0.0)
  o_ref[...] = jnp.dot(h1, w2_ref[...],
                       preferred_element_type=jnp.float32) + b2_ref[...]


def _head(h, fc1_w, fc1_b, fc2_w, fc2_b):
  return pl.pallas_call(
      _head_body,
      out_shape=jax.ShapeDtypeStruct((1, D), jnp.float32),
  )(h, fc1_w, fc1_b.reshape(1, -1), fc2_w, fc2_b.reshape(1, D))


def kernel(x, edge_index, W_msg0, W_self0, b0, W_msg1, W_self1, b1,
           fc1_w, fc1_b, fc2_w, fc2_b):
  src = edge_index[0].astype(jnp.int32).reshape(NW, CH, K)
  dst = edge_index[1].astype(jnp.int32).reshape(NW, CH, K)

  x_aug = jnp.concatenate(
      [x, jnp.ones((N, 1), jnp.float32), jnp.zeros((N, DA - D - 1), jnp.float32)],
      axis=1)
  P1 = _sc_segsum_aug(x_aug, src, dst).reshape(2, NP, DA)[:, :N]
  degT = jnp.transpose(P1[:, :, D])
  h1 = _dense_layer(P1[:, :, :D], degT, x, W_msg0, W_self0, b0)

  P2 = _sc_segsum(h1, src, dst).reshape(2, NP, D)[:, :N]
  h2 = _dense_layer(P2, degT, h1, W_msg1, W_self1, b1)

  return _head(h2, fc1_w, fc1_b, fc2_w, fc2_b)


# speedup vs baseline: 7.4822x; 1.1749x over previous
import functools

import jax
import jax.numpy as jnp
from jax import lax
from jax.experimental import pallas as pl
from jax.experimental.pallas import tpu as pltpu
from jax.experimental.pallas import tpu_sc as plsc

N = 10000
E = 320000
D = 128
DA = 144

NC = 2
NS = 16
NW = NC * NS
EW = E // NW
K = 100
CH = EW // K
IB = 20
NBLK = CH // IB
NP = 10240
RPT = NP // NS
RZC = 80


def _make_sc_segsum(width):
  mesh = plsc.VectorSubcoreMesh(core_axis_name="c", subcore_axis_name="s")

  @functools.partial(
      pl.kernel,
      out_type=jax.ShapeDtypeStruct((2 * NP, width), jnp.float32),
      mesh=mesh,
      compiler_params=pltpu.CompilerParams(use_tc_tiling_on_sc=False),
      scratch_types=[
          pltpu.VMEM((IB, K), jnp.int32),
          pltpu.VMEM((IB, K), jnp.int32),
          pltpu.VMEM((K, width), jnp.float32),
          pltpu.VMEM((K, width), jnp.float32),
          pltpu.VMEM_SHARED((NP, width), jnp.float32),
          pltpu.SemaphoreType.DMA,
          pltpu.SemaphoreType.DMA,
          pltpu.SemaphoreType.DMA,
          pltpu.SemaphoreType.DMA,
      ],
  )
  def segsum(tab_hbm, src_hbm, dst_hbm, out_hbm,
             src_v, dst_v, rows0, rows1, acc_sh, sg0, sg1, ss0, ss1):
    cid = lax.axis_index("c")
    sid = lax.axis_index("s")
    wid = cid * NS + sid

    zero = jnp.zeros((16,), jnp.float32)

    def zrow(r, carry):
      def zcol(c, carry2):
        rows0[r, pl.ds(c * 16, 16)] = zero
        return carry2
      return lax.fori_loop(0, width // 16, zcol, carry)

    lax.fori_loop(0, RZC, zrow, 0)
    for j in range(RPT // RZC):
      pltpu.sync_copy(rows0.at[pl.ds(0, RZC)],
                      acc_sh.at[pl.ds(sid * RPT + j * RZC, RZC)])

    plsc.subcore_barrier()

    def g_issue(rows, c, sem):
      pltpu.async_copy(tab_hbm.at[src_v.at[c]], rows, sem)

    def g_wait(rows, c, sem):
      pltpu.make_async_copy(tab_hbm.at[src_v.at[c]], rows, sem).wait()

    def s_issue(rows, c, sem):
      pltpu.async_copy(rows, acc_sh.at[dst_v.at[c]], sem, add=True)

    def s_wait(rows, c, sem):
      pltpu.make_async_copy(rows, acc_sh.at[dst_v.at[c]], sem).wait()

    for b in range(NBLK):
      pltpu.sync_copy(src_hbm.at[wid, pl.ds(b * IB, IB)], src_v)
      pltpu.sync_copy(dst_hbm.at[wid, pl.ds(b * IB, IB)], dst_v)

      g_issue(rows0, 0, sg0)
      g_issue(rows1, 1, sg1)

      def pair(p, carry):
        c0 = 2 * p
        c1 = c0 + 1
        g_wait(rows0, c0, sg0)
        s_issue(rows0, c0, ss0)
        g_wait(rows1, c1, sg1)
        s_issue(rows1, c1, ss1)
        s_wait(rows0, c0, ss0)
        g_issue(rows0, c0 + 2, sg0)
        s_wait(rows1, c1, ss1)
        g_issue(rows1, c1 + 2, sg1)
        return carry

      lax.fori_loop(0, IB // 2 - 1, pair, 0)

      c0, c1 = IB - 2, IB - 1
      g_wait(rows0, c0, sg0)
      s_issue(rows0, c0, ss0)
      g_wait(rows1, c1, sg1)
      s_issue(rows1, c1, ss1)
      s_wait(rows0, c0, ss0)
      s_wait(rows1, c1, ss1)

    plsc.subcore_barrier()

    for j in range(RPT // RZC):
      row = sid * RPT + j * RZC
      pltpu.sync_copy(acc_sh.at[pl.ds(row, RZC)], rows0.at[pl.ds(0, RZC)])
      pltpu.sync_copy(rows0.at[pl.ds(0, RZC)],
                      out_hbm.at[pl.ds(cid * NP + row, RZC)])

  return segsum


_sc_segsum_aug = _make_sc_segsum(DA)
_sc_segsum = _make_sc_segsum(D)


_RB = 2000


def _dense_body(p_ref, d_ref, x_ref, wm_ref, ws_ref, b_ref, o_ref):
  a = p_ref[0] + p_ref[1]
  deg = d_ref[:, 0:1] + d_ref[:, 1:2]
  scale = 1.0 / jnp.maximum(deg, 1.0)
  agg = jnp.dot(a * scale, wm_ref[...], preferred_element_type=jnp.float32)
  self_t = jnp.dot(x_ref[...], ws_ref[...], preferred_element_type=jnp.float32)
  o_ref[...] = jnp.maximum(agg + self_t + b_ref[...], 0.0)


def _dense_layer(P, degT, x, Wm, Ws, b):
  return pl.pallas_call(
      _dense_body,
      grid=(N // _RB,),
      in_specs=[
          pl.BlockSpec((2, _RB, D), lambda i: (0, i, 0)),
          pl.BlockSpec((_RB, 2), lambda i: (i, 0)),
          pl.BlockSpec((_RB, D), lambda i: (i, 0)),
          pl.BlockSpec((D, D), lambda i: (0, 0)),
          pl.BlockSpec((D, D), lambda i: (0, 0)),
          pl.BlockSpec((1, D), lambda i: (0, 0)),
      ],
      out_specs=pl.BlockSpec((_RB, D), lambda i: (i, 0)),
      out_shape=jax.ShapeDtypeStruct((N, D), jnp.float32),
  )(P, degT, x, Wm, Ws, b.reshape(1, D))


def _head_body(h_ref, w1_ref, b1_ref, w2_ref, b2_ref, o_ref):
  g = jnp.max(h_ref[...], axis=0, keepdims=True)
  h1 = jnp.maximum(
      jnp.dot(g, w1_ref[...], preferred_element_type=jnp.float32) + b1_ref[...],
      0.0)
  o_ref[...] = jnp.dot(h1, w2_ref[...],
                       preferred_element_type=jnp.float32) + b2_ref[...]


def _head(h, fc1_w, fc1_b, fc2_w, fc2_b):
  return pl.pallas_call(
      _head_body,
      out_shape=jax.ShapeDtypeStruct((1, D), jnp.float32),
  )(h, fc1_w, fc1_b.reshape(1, -1), fc2_w, fc2_b.reshape(1, D))


def kernel(x, edge_index, W_msg0, W_self0, b0, W_msg1, W_self1, b1,
           fc1_w, fc1_b, fc2_w, fc2_b):
  src = edge_index[0].astype(jnp.int32).reshape(NW, CH, K)
  dst = edge_index[1].astype(jnp.int32).reshape(NW, CH, K)

  x_aug = jnp.concatenate(
      [x, jnp.ones((N, 1), jnp.float32), jnp.zeros((N, DA - D - 1), jnp.float32)],
      axis=1)
  P1 = _sc_segsum_aug(x_aug, src, dst).reshape(2, NP, DA)[:, :N]
  degT = jnp.transpose(P1[:, :, D])
  h1 = _dense_layer(P1[:, :, :D], degT, x, W_msg0, W_self0, b0)

  P2 = _sc_segsum(h1, src, dst).reshape(2, NP, D)[:, :N]
  h2 = _dense_layer(P2, degT, h1, W_msg1, W_self1, b1)

  return _head(h2, fc1_w, fc1_b, fc2_w, fc2_b)


# speedup vs baseline: 10.1606x; 1.3580x over previous
import functools

import jax
import jax.numpy as jnp
from jax import lax
from jax.experimental import pallas as pl
from jax.experimental.pallas import tpu as pltpu
from jax.experimental.pallas import tpu_sc as plsc

N = 10000
E = 320000
D = 128

NC = 2
NS = 16
NW = NC * NS
EW = E // NW
K = 100
CH = EW // K
IB = 20
NBLK = CH // IB
NP = 10240
RPT = NP // NS
RZC = 80
KF = K // 16
KR = K - 16 * KF


def _make_sc_segsum(with_deg):
  mesh = plsc.VectorSubcoreMesh(core_axis_name="c", subcore_axis_name="s")

  out_type = jax.ShapeDtypeStruct((2 * NP, D), jnp.float32)
  scratch = [
      pltpu.VMEM((IB, K), jnp.int32),
      pltpu.VMEM((IB, K), jnp.int32),
      pltpu.VMEM((K, D), jnp.float32),
      pltpu.VMEM((K, D), jnp.float32),
      pltpu.VMEM_SHARED((NP, D), jnp.float32),
      pltpu.SemaphoreType.DMA,
      pltpu.SemaphoreType.DMA,
      pltpu.SemaphoreType.DMA,
      pltpu.SemaphoreType.DMA,
  ]
  if with_deg:
    out_type = [out_type, jax.ShapeDtypeStruct((NW, NP), jnp.float32)]
    scratch = scratch + [
        pltpu.VMEM((NP,), jnp.float32),
    ]

  @functools.partial(
      pl.kernel,
      out_type=out_type,
      mesh=mesh,
      compiler_params=pltpu.CompilerParams(use_tc_tiling_on_sc=False,
                                           needs_layout_passes=False),
      scratch_types=scratch,
  )
  def segsum(tab_hbm, src_hbm, dst_hbm, out_hbm, *rest):
    if with_deg:
      (deg_hbm, src_v, dst_v, rows0, rows1, acc_sh,
       sg0, sg1, ss0, ss1, degloc) = rest
    else:
      src_v, dst_v, rows0, rows1, acc_sh, sg0, sg1, ss0, ss1 = rest
    cid = lax.axis_index("c")
    sid = lax.axis_index("s")
    wid = cid * NS + sid

    zero = jnp.zeros((16,), jnp.float32)

    def zrow(r, carry):
      def zcol(c, carry2):
        rows0[r, pl.ds(c * 16, 16)] = zero
        return carry2
      return lax.fori_loop(0, D // 16, zcol, carry)

    lax.fori_loop(0, RZC, zrow, 0)
    for j in range(RPT // RZC):
      pltpu.sync_copy(rows0.at[pl.ds(0, RZC)],
                      acc_sh.at[pl.ds(sid * RPT + j * RZC, RZC)])

    if with_deg:
      def zdeg(r, carry):
        degloc[pl.ds(r * 16, 16)] = zero
        return carry
      lax.fori_loop(0, NP // 16, zdeg, 0)

    plsc.subcore_barrier()

    def g_issue(rows, c, sem):
      pltpu.async_copy(tab_hbm.at[src_v.at[c]], rows, sem)

    def g_wait(rows, c, sem):
      pltpu.make_async_copy(tab_hbm.at[src_v.at[c]], rows, sem).wait()

    def s_issue(rows, c, sem):
      pltpu.async_copy(rows, acc_sh.at[dst_v.at[c]], sem, add=True)

    def s_wait(rows, c, sem):
      pltpu.make_async_copy(rows, acc_sh.at[dst_v.at[c]], sem).wait()

    if with_deg:
      ones16 = jnp.ones((16,), jnp.float32)
      rmask = lax.iota(jnp.int32, 16) >= 16 - KR

      def histo(c):
        for g in range(KF):
          idx = dst_v[c, pl.ds(g * 16, 16)]
          plsc.addupdate_scatter(degloc, [idx], ones16)
        if KR:
          idx = dst_v[c, pl.ds(K - 16, 16)]
          plsc.addupdate_scatter(degloc, [idx], ones16, mask=rmask)
    else:
      def histo(c):
        del c

    for b in range(NBLK):
      pltpu.sync_copy(src_hbm.at[wid, pl.ds(b * IB, IB)], src_v)
      pltpu.sync_copy(dst_hbm.at[wid, pl.ds(b * IB, IB)], dst_v)

      g_issue(rows0, 0, sg0)
      g_issue(rows1, 1, sg1)

      def pair(p, carry):
        c0 = 2 * p
        c1 = c0 + 1
        g_wait(rows0, c0, sg0)
        s_issue(rows0, c0, ss0)
        g_wait(rows1, c1, sg1)
        s_issue(rows1, c1, ss1)
        histo(c0)
        histo(c1)
        s_wait(rows0, c0, ss0)
        g_issue(rows0, c0 + 2, sg0)
        s_wait(rows1, c1, ss1)
        g_issue(rows1, c1 + 2, sg1)
        return carry

      lax.fori_loop(0, IB // 2 - 1, pair, 0)

      c0, c1 = IB - 2, IB - 1
      g_wait(rows0, c0, sg0)
      s_issue(rows0, c0, ss0)
      g_wait(rows1, c1, sg1)
      s_issue(rows1, c1, ss1)
      histo(c0)
      histo(c1)
      s_wait(rows0, c0, ss0)
      s_wait(rows1, c1, ss1)

    if with_deg:
      pltpu.sync_copy(degloc, deg_hbm.at[wid])

    plsc.subcore_barrier()

    for j in range(RPT // RZC):
      row = sid * RPT + j * RZC
      pltpu.sync_copy(acc_sh.at[pl.ds(row, RZC)], rows0.at[pl.ds(0, RZC)])
      pltpu.sync_copy(rows0.at[pl.ds(0, RZC)],
                      out_hbm.at[pl.ds(cid * NP + row, RZC)])

  return segsum


_sc_segsum_deg = _make_sc_segsum(True)
_sc_segsum = _make_sc_segsum(False)


_RB = 2048


def _dense_body(p_ref, d_ref, x_ref, wm_ref, ws_ref, b_ref, o_ref):
  a = p_ref[0] + p_ref[1]
  deg = jnp.sum(d_ref[...], axis=1, keepdims=True)
  scale = 1.0 / jnp.maximum(deg, 1.0)
  agg = jnp.dot(a * scale, wm_ref[...], preferred_element_type=jnp.float32)
  self_t = jnp.dot(x_ref[...], ws_ref[...], preferred_element_type=jnp.float32)
  o_ref[...] = jnp.maximum(agg + self_t + b_ref[...], 0.0)


def _dense_layer(P, degT, x, Wm, Ws, b):
  return pl.pallas_call(
      _dense_body,
      grid=(NP // _RB,),
      in_specs=[
          pl.BlockSpec((2, _RB, D), lambda i: (0, i, 0)),
          pl.BlockSpec((_RB, NW), lambda i: (i, 0)),
          pl.BlockSpec((_RB, D), lambda i: (i, 0)),
          pl.BlockSpec((D, D), lambda i: (0, 0)),
          pl.BlockSpec((D, D), lambda i: (0, 0)),
          pl.BlockSpec((1, D), lambda i: (0, 0)),
      ],
      out_specs=pl.BlockSpec((_RB, D), lambda i: (i, 0)),
      out_shape=jax.ShapeDtypeStruct((NP, D), jnp.float32),
  )(P, degT, x, Wm, Ws, b.reshape(1, D))


def _head_body(h_ref, w1_ref, b1_ref, w2_ref, b2_ref, o_ref):
  rows = lax.broadcasted_iota(jnp.int32, (NP, 1), 0)
  hv = jnp.where(rows < N, h_ref[...], -jnp.inf)
  g = jnp.max(hv, axis=0, keepdims=True)
  h1 = jnp.maximum(
      jnp.dot(g, w1_ref[...], preferred_element_type=jnp.float32) + b1_ref[...],
      0.0)
  o_ref[...] = jnp.dot(h1, w2_ref[...],
                       preferred_element_type=jnp.float32) + b2_ref[...]


def _head(h, fc1_w, fc1_b, fc2_w, fc2_b):
  return pl.pallas_call(
      _head_body,
      out_shape=jax.ShapeDtypeStruct((1, D), jnp.float32),
  )(h, fc1_w, fc1_b.reshape(1, -1), fc2_w, fc2_b.reshape(1, D))


def kernel(x, edge_index, W_msg0, W_self0, b0, W_msg1, W_self1, b1,
           fc1_w, fc1_b, fc2_w, fc2_b):
  src = edge_index[0].astype(jnp.int32).reshape(NW, CH, K)
  dst = edge_index[1].astype(jnp.int32).reshape(NW, CH, K)

  x_pad = jnp.pad(x, ((0, NP - N), (0, 0)))

  P1, deg = _sc_segsum_deg(x_pad, src, dst)
  degT = jnp.transpose(deg)
  h1 = _dense_layer(P1.reshape(2, NP, D), degT, x_pad, W_msg0, W_self0, b0)

  P2 = _sc_segsum(h1, src, dst)
  h2 = _dense_layer(P2.reshape(2, NP, D), degT, h1, W_msg1, W_self1, b1)

  return _head(h2, fc1_w, fc1_b, fc2_w, fc2_b)


# speedup vs baseline: 10.3181x; 1.0155x over previous
import functools

import jax
import jax.numpy as jnp
from jax import lax
from jax.experimental import pallas as pl
from jax.experimental.pallas import tpu as pltpu
from jax.experimental.pallas import tpu_sc as plsc

N = 10000
E = 320000
D = 128

NC = 2
NS = 16
NW = NC * NS
EW = E // NW
K = 100
CH = EW // K
IB = 20
NBLK = CH // IB
NP = 10240
RPT = NP // NS
RZC = 80
KF = K // 16
KR = K - 16 * KF


def _make_sc_segsum(with_deg):
  mesh = plsc.VectorSubcoreMesh(core_axis_name="c", subcore_axis_name="s")

  out_type = jax.ShapeDtypeStruct((2 * NP, D), jnp.float32)
  scratch = [
      pltpu.VMEM((IB, K), jnp.int32),
      pltpu.VMEM((IB, K), jnp.int32),
      pltpu.VMEM((K, D), jnp.float32),
      pltpu.VMEM((K, D), jnp.float32),
      pltpu.VMEM_SHARED((NP, D), jnp.float32),
      pltpu.SemaphoreType.DMA,
      pltpu.SemaphoreType.DMA,
      pltpu.SemaphoreType.DMA,
      pltpu.SemaphoreType.DMA,
  ]
  if with_deg:
    out_type = [out_type, jax.ShapeDtypeStruct((NW, NP), jnp.float32)]
    scratch = scratch + [
        pltpu.VMEM((NP,), jnp.float32),
    ]

  @functools.partial(
      pl.kernel,
      out_type=out_type,
      mesh=mesh,
      compiler_params=pltpu.CompilerParams(use_tc_tiling_on_sc=False,
                                           needs_layout_passes=False),
      scratch_types=scratch,
  )
  def segsum(tab_hbm, src_hbm, dst_hbm, out_hbm, *rest):
    if with_deg:
      (deg_hbm, src_v, dst_v, rows0, rows1, acc_sh,
       sg0, sg1, ss0, ss1, degloc) = rest
    else:
      src_v, dst_v, rows0, rows1, acc_sh, sg0, sg1, ss0, ss1 = rest
    cid = lax.axis_index("c")
    sid = lax.axis_index("s")
    wid = cid * NS + sid

    zero = jnp.zeros((16,), jnp.float32)

    def zrow(r, carry):
      def zcol(c, carry2):
        rows0[r, pl.ds(c * 16, 16)] = zero
        return carry2
      return lax.fori_loop(0, D // 16, zcol, carry)

    lax.fori_loop(0, RZC, zrow, 0)
    for j in range(RPT // RZC):
      pltpu.sync_copy(rows0.at[pl.ds(0, RZC)],
                      acc_sh.at[pl.ds(sid * RPT + j * RZC, RZC)])

    if with_deg:
      def zdeg(r, carry):
        degloc[pl.ds(r * 16, 16)] = zero
        return carry
      lax.fori_loop(0, NP // 16, zdeg, 0)

    plsc.subcore_barrier()

    def g_issue(rows, c, sem):
      pltpu.async_copy(tab_hbm.at[src_v.at[c]], rows, sem)

    def g_wait(rows, c, sem):
      pltpu.make_async_copy(tab_hbm.at[src_v.at[c]], rows, sem).wait()

    def s_issue(rows, c, sem):
      pltpu.async_copy(rows, acc_sh.at[dst_v.at[c]], sem, add=True)

    def s_wait(rows, c, sem):
      pltpu.make_async_copy(rows, acc_sh.at[dst_v.at[c]], sem).wait()

    if with_deg:
      ones16 = jnp.ones((16,), jnp.float32)
      rmask = lax.iota(jnp.int32, 16) >= 16 - KR

      def histo(c):
        for g in range(KF):
          idx = dst_v[c, pl.ds(g * 16, 16)]
          plsc.addupdate_scatter(degloc, [idx], ones16)
        if KR:
          idx = dst_v[c, pl.ds(K - 16, 16)]
          plsc.addupdate_scatter(degloc, [idx], ones16, mask=rmask)
    else:
      def histo(c):
        del c

    for b in range(NBLK):
      pltpu.sync_copy(src_hbm.at[wid, pl.ds(b * IB, IB)], src_v)
      pltpu.sync_copy(dst_hbm.at[wid, pl.ds(b * IB, IB)], dst_v)

      g_issue(rows0, 0, sg0)
      g_issue(rows1, 1, sg1)

      def pair(p, carry):
        c0 = 2 * p
        c1 = c0 + 1
        g_wait(rows0, c0, sg0)
        s_issue(rows0, c0, ss0)
        g_wait(rows1, c1, sg1)
        s_issue(rows1, c1, ss1)
        histo(c0)
        histo(c1)
        s_wait(rows0, c0, ss0)
        g_issue(rows0, c0 + 2, sg0)
        s_wait(rows1, c1, ss1)
        g_issue(rows1, c1 + 2, sg1)
        return carry

      lax.fori_loop(0, IB // 2 - 1, pair, 0)

      c0, c1 = IB - 2, IB - 1
      g_wait(rows0, c0, sg0)
      s_issue(rows0, c0, ss0)
      g_wait(rows1, c1, sg1)
      s_issue(rows1, c1, ss1)
      histo(c0)
      histo(c1)
      s_wait(rows0, c0, ss0)
      s_wait(rows1, c1, ss1)

    if with_deg:
      pltpu.sync_copy(degloc, deg_hbm.at[wid])

    plsc.subcore_barrier()

    for j in range(RPT // RZC):
      row = sid * RPT + j * RZC
      pltpu.sync_copy(acc_sh.at[pl.ds(row, RZC)], rows0.at[pl.ds(0, RZC)])
      pltpu.sync_copy(rows0.at[pl.ds(0, RZC)],
                      out_hbm.at[pl.ds(cid * NP + row, RZC)])

  return segsum


_sc_segsum_deg = _make_sc_segsum(True)
_sc_segsum = _make_sc_segsum(False)


_RB = 2048


def _dense_body(p_ref, d_ref, x_ref, wm_ref, ws_ref, b_ref, o_ref):
  a = p_ref[0] + p_ref[1]
  deg = jnp.sum(d_ref[...], axis=1, keepdims=True)
  scale = 1.0 / jnp.maximum(deg, 1.0)
  agg = jnp.dot(a * scale, wm_ref[...], preferred_element_type=jnp.float32)
  self_t = jnp.dot(x_ref[...], ws_ref[...], preferred_element_type=jnp.float32)
  o_ref[...] = jnp.maximum(agg + self_t + b_ref[...], 0.0)


def _dense_layer(P, degT, x, Wm, Ws, b):
  return pl.pallas_call(
      _dense_body,
      grid=(NP // _RB,),
      in_specs=[
          pl.BlockSpec((2, _RB, D), lambda i: (0, i, 0)),
          pl.BlockSpec((_RB, NW), lambda i: (i, 0)),
          pl.BlockSpec((_RB, D), lambda i: (i, 0)),
          pl.BlockSpec((D, D), lambda i: (0, 0)),
          pl.BlockSpec((D, D), lambda i: (0, 0)),
          pl.BlockSpec((1, D), lambda i: (0, 0)),
      ],
      out_specs=pl.BlockSpec((_RB, D), lambda i: (i, 0)),
      out_shape=jax.ShapeDtypeStruct((NP, D), jnp.float32),
  )(P, degT, x, Wm, Ws, b.reshape(1, D))


def _dense2_head_body(p_ref, d_ref, x_ref, wm_ref, ws_ref, b_ref,
                      w1_ref, b1_ref, w2_ref, b2_ref, o_ref, gmax):
  i = pl.program_id(0)
  a = p_ref[0] + p_ref[1]
  deg = jnp.sum(d_ref[...], axis=1, keepdims=True)
  scale = 1.0 / jnp.maximum(deg, 1.0)
  agg = jnp.dot(a * scale, wm_ref[...], preferred_element_type=jnp.float32)
  self_t = jnp.dot(x_ref[...], ws_ref[...], preferred_element_type=jnp.float32)
  h2 = jnp.maximum(agg + self_t + b_ref[...], 0.0)
  rows = i * _RB + lax.broadcasted_iota(jnp.int32, (_RB, 1), 0)
  bm = jnp.max(jnp.where(rows < N, h2, -jnp.inf), axis=0, keepdims=True)

  @pl.when(i == 0)
  def _():
    gmax[...] = bm

  @pl.when(i > 0)
  def _():
    gmax[...] = jnp.maximum(gmax[...], bm)

  @pl.when(i == NP // _RB - 1)
  def _():
    g = gmax[...]
    h1v = jnp.maximum(
        jnp.dot(g, w1_ref[...], preferred_element_type=jnp.float32)
        + b1_ref[...], 0.0)
    o_ref[...] = jnp.dot(h1v, w2_ref[...],
                         preferred_element_type=jnp.float32) + b2_ref[...]


def _dense2_head(P, degT, x, Wm, Ws, b, fc1_w, fc1_b, fc2_w, fc2_b):
  return pl.pallas_call(
      _dense2_head_body,
      grid=(NP // _RB,),
      in_specs=[
          pl.BlockSpec((2, _RB, D), lambda i: (0, i, 0)),
          pl.BlockSpec((_RB, NW), lambda i: (i, 0)),
          pl.BlockSpec((_RB, D), lambda i: (i, 0)),
          pl.BlockSpec((D, D), lambda i: (0, 0)),
          pl.BlockSpec((D, D), lambda i: (0, 0)),
          pl.BlockSpec((1, D), lambda i: (0, 0)),
          pl.BlockSpec((D, D // 2), lambda i: (0, 0)),
          pl.BlockSpec((1, D // 2), lambda i: (0, 0)),
          pl.BlockSpec((D // 2, D), lambda i: (0, 0)),
          pl.BlockSpec((1, D), lambda i: (0, 0)),
      ],
      out_specs=pl.BlockSpec((1, D), lambda i: (0, 0)),
      out_shape=jax.ShapeDtypeStruct((1, D), jnp.float32),
      scratch_shapes=[pltpu.VMEM((1, D), jnp.float32)],
  )(P, degT, x, Wm, Ws, b.reshape(1, D),
    fc1_w, fc1_b.reshape(1, -1), fc2_w, fc2_b.reshape(1, D))


def kernel(x, edge_index, W_msg0, W_self0, b0, W_msg1, W_self1, b1,
           fc1_w, fc1_b, fc2_w, fc2_b):
  src = edge_index[0].astype(jnp.int32).reshape(NW, CH, K)
  dst = edge_index[1].astype(jnp.int32).reshape(NW, CH, K)

  x_pad = jnp.pad(x, ((0, NP - N), (0, 0)))

  P1, deg = _sc_segsum_deg(x_pad, src, dst)
  degT = jnp.transpose(deg)
  h1 = _dense_layer(P1.reshape(2, NP, D), degT, x_pad, W_msg0, W_self0, b0)

  P2 = _sc_segsum(h1, src, dst)
  return _dense2_head(P2.reshape(2, NP, D), degT, h1, W_msg1, W_self1, b1,
                      fc1_w, fc1_b, fc2_w, fc2_b)


# speedup vs baseline: 10.8905x; 1.0555x over previous
import functools

import jax
import jax.numpy as jnp
from jax import lax
from jax.experimental import pallas as pl
from jax.experimental.pallas import tpu as pltpu
from jax.experimental.pallas import tpu_sc as plsc

N = 10000
E = 320000
D = 128

NC = 2
NS = 16
NW = NC * NS
EW = E // NW
K = 50
CH = EW // K
IB = 20
NBLK = CH // IB
NP = 10240
RPT = NP // NS
RZC = 40
KF = K // 16
KR = K - 16 * KF


def _make_sc_segsum(with_deg):
  mesh = plsc.VectorSubcoreMesh(core_axis_name="c", subcore_axis_name="s")

  out_type = jax.ShapeDtypeStruct((2 * NP, D), jnp.float32)
  scratch = [
      pltpu.VMEM((IB, K), jnp.int32),
      pltpu.VMEM((IB, K), jnp.int32),
      pltpu.VMEM((K, D), jnp.float32),
      pltpu.VMEM((K, D), jnp.float32),
      pltpu.VMEM((K, D), jnp.float32),
      pltpu.VMEM((K, D), jnp.float32),
      pltpu.VMEM_SHARED((NP, D), jnp.float32),
      pltpu.SemaphoreType.DMA,
      pltpu.SemaphoreType.DMA,
      pltpu.SemaphoreType.DMA,
      pltpu.SemaphoreType.DMA,
      pltpu.SemaphoreType.DMA,
      pltpu.SemaphoreType.DMA,
      pltpu.SemaphoreType.DMA,
      pltpu.SemaphoreType.DMA,
  ]
  if with_deg:
    out_type = [out_type, jax.ShapeDtypeStruct((NW, NP), jnp.float32)]
    scratch = scratch + [
        pltpu.VMEM((NP,), jnp.float32),
    ]

  @functools.partial(
      pl.kernel,
      out_type=out_type,
      mesh=mesh,
      compiler_params=pltpu.CompilerParams(use_tc_tiling_on_sc=False,
                                           needs_layout_passes=False),
      scratch_types=scratch,
  )
  def segsum(tab_hbm, src_hbm, dst_hbm, out_hbm, *rest):
    if with_deg:
      (deg_hbm, src_v, dst_v, rows0, rows1, rows2, rows3, acc_sh,
       sg0, sg1, sg2, sg3, ss0, ss1, ss2, ss3, degloc) = rest
    else:
      (src_v, dst_v, rows0, rows1, rows2, rows3, acc_sh,
       sg0, sg1, sg2, sg3, ss0, ss1, ss2, ss3) = rest
    slots = ((rows0, sg0, ss0), (rows1, sg1, ss1),
             (rows2, sg2, ss2), (rows3, sg3, ss3))
    cid = lax.axis_index("c")
    sid = lax.axis_index("s")
    wid = cid * NS + sid

    zero = jnp.zeros((16,), jnp.float32)

    def zrow(r, carry):
      def zcol(c, carry2):
        rows0[r, pl.ds(c * 16, 16)] = zero
        return carry2
      return lax.fori_loop(0, D // 16, zcol, carry)

    lax.fori_loop(0, RZC, zrow, 0)
    for j in range(RPT // RZC):
      pltpu.sync_copy(rows0.at[pl.ds(0, RZC)],
                      acc_sh.at[pl.ds(sid * RPT + j * RZC, RZC)])

    if with_deg:
      def zdeg(r, carry):
        degloc[pl.ds(r * 16, 16)] = zero
        return carry
      lax.fori_loop(0, NP // 16, zdeg, 0)

    plsc.subcore_barrier()

    def g_issue(rows, c, sem):
      pltpu.async_copy(tab_hbm.at[src_v.at[c]], rows, sem)

    def g_wait(rows, c, sem):
      pltpu.make_async_copy(tab_hbm.at[src_v.at[c]], rows, sem).wait()

    def s_issue(rows, c, sem):
      pltpu.async_copy(rows, acc_sh.at[dst_v.at[c]], sem, add=True)

    def s_wait(rows, c, sem):
      pltpu.make_async_copy(rows, acc_sh.at[dst_v.at[c]], sem).wait()

    if with_deg:
      ones16 = jnp.ones((16,), jnp.float32)
      rmask = lax.iota(jnp.int32, 16) >= 16 - KR

      def histo(c):
        for g in range(KF):
          idx = dst_v[c, pl.ds(g * 16, 16)]
          plsc.addupdate_scatter(degloc, [idx], ones16)
        if KR:
          idx = dst_v[c, pl.ds(K - 16, 16)]
          plsc.addupdate_scatter(degloc, [idx], ones16, mask=rmask)
    else:
      def histo(c):
        del c

    for b in range(NBLK):
      pltpu.sync_copy(src_hbm.at[wid, pl.ds(b * IB, IB)], src_v)
      pltpu.sync_copy(dst_hbm.at[wid, pl.ds(b * IB, IB)], dst_v)

      for t, (rows, sg, ss) in enumerate(slots):
        g_issue(rows, t, sg)

      def quad(q, carry):
        c0 = 4 * q
        for t, (rows, sg, ss) in enumerate(slots):
          g_wait(rows, c0 + t, sg)
          s_issue(rows, c0 + t, ss)
          histo(c0 + t)
        for t, (rows, sg, ss) in enumerate(slots):
          s_wait(rows, c0 + t, ss)
          g_issue(rows, c0 + t + 4, sg)
        return carry

      lax.fori_loop(0, IB // 4 - 1, quad, 0)

      c0 = IB - 4
      for t, (rows, sg, ss) in enumerate(slots):
        g_wait(rows, c0 + t, sg)
        s_issue(rows, c0 + t, ss)
        histo(c0 + t)
      for t, (rows, sg, ss) in enumerate(slots):
        s_wait(rows, c0 + t, ss)

    if with_deg:
      pltpu.sync_copy(degloc, deg_hbm.at[wid])

    plsc.subcore_barrier()

    for j in range(RPT // RZC):
      row = sid * RPT + j * RZC
      pltpu.sync_copy(acc_sh.at[pl.ds(row, RZC)], rows0.at[pl.ds(0, RZC)])
      pltpu.sync_copy(rows0.at[pl.ds(0, RZC)],
                      out_hbm.at[pl.ds(cid * NP + row, RZC)])

  return segsum


_sc_segsum_deg = _make_sc_segsum(True)
_sc_segsum = _make_sc_segsum(False)


_RB = 2048


def _dense_body(p_ref, d_ref, x_ref, wm_ref, ws_ref, b_ref, o_ref):
  a = p_ref[0] + p_ref[1]
  deg = jnp.sum(d_ref[...], axis=1, keepdims=True)
  scale = 1.0 / jnp.maximum(deg, 1.0)
  agg = jnp.dot(a * scale, wm_ref[...], preferred_element_type=jnp.float32)
  self_t = jnp.dot(x_ref[...], ws_ref[...], preferred_element_type=jnp.float32)
  o_ref[...] = jnp.maximum(agg + self_t + b_ref[...], 0.0)


def _dense_layer(P, degT, x, Wm, Ws, b):
  return pl.pallas_call(
      _dense_body,
      grid=(NP // _RB,),
      in_specs=[
          pl.BlockSpec((2, _RB, D), lambda i: (0, i, 0)),
          pl.BlockSpec((_RB, NW), lambda i: (i, 0)),
          pl.BlockSpec((_RB, D), lambda i: (i, 0)),
          pl.BlockSpec((D, D), lambda i: (0, 0)),
          pl.BlockSpec((D, D), lambda i: (0, 0)),
          pl.BlockSpec((1, D), lambda i: (0, 0)),
      ],
      out_specs=pl.BlockSpec((_RB, D), lambda i: (i, 0)),
      out_shape=jax.ShapeDtypeStruct((NP, D), jnp.float32),
  )(P, degT, x, Wm, Ws, b.reshape(1, D))


def _dense2_head_body(p_ref, d_ref, x_ref, wm_ref, ws_ref, b_ref,
                      w1_ref, b1_ref, w2_ref, b2_ref, o_ref, gmax):
  i = pl.program_id(0)
  a = p_ref[0] + p_ref[1]
  deg = jnp.sum(d_ref[...], axis=1, keepdims=True)
  scale = 1.0 / jnp.maximum(deg, 1.0)
  agg = jnp.dot(a * scale, wm_ref[...], preferred_element_type=jnp.float32)
  self_t = jnp.dot(x_ref[...], ws_ref[...], preferred_element_type=jnp.float32)
  h2 = jnp.maximum(agg + self_t + b_ref[...], 0.0)
  rows = i * _RB + lax.broadcasted_iota(jnp.int32, (_RB, 1), 0)
  bm = jnp.max(jnp.where(rows < N, h2, -jnp.inf), axis=0, keepdims=True)

  @pl.when(i == 0)
  def _():
    gmax[...] = bm

  @pl.when(i > 0)
  def _():
    gmax[...] = jnp.maximum(gmax[...], bm)

  @pl.when(i == NP // _RB - 1)
  def _():
    g = gmax[...]
    h1v = jnp.maximum(
        jnp.dot(g, w1_ref[...], preferred_element_type=jnp.float32)
        + b1_ref[...], 0.0)
    o_ref[...] = jnp.dot(h1v, w2_ref[...],
                         preferred_element_type=jnp.float32) + b2_ref[...]


def _dense2_head(P, degT, x, Wm, Ws, b, fc1_w, fc1_b, fc2_w, fc2_b):
  return pl.pallas_call(
      _dense2_head_body,
      grid=(NP // _RB,),
      in_specs=[
          pl.BlockSpec((2, _RB, D), lambda i: (0, i, 0)),
          pl.BlockSpec((_RB, NW), lambda i: (i, 0)),
          pl.BlockSpec((_RB, D), lambda i: (i, 0)),
          pl.BlockSpec((D, D), lambda i: (0, 0)),
          pl.BlockSpec((D, D), lambda i: (0, 0)),
          pl.BlockSpec((1, D), lambda i: (0, 0)),
          pl.BlockSpec((D, D // 2), lambda i: (0, 0)),
          pl.BlockSpec((1, D // 2), lambda i: (0, 0)),
          pl.BlockSpec((D // 2, D), lambda i: (0, 0)),
          pl.BlockSpec((1, D), lambda i: (0, 0)),
      ],
      out_specs=pl.BlockSpec((1, D), lambda i: (0, 0)),
      out_shape=jax.ShapeDtypeStruct((1, D), jnp.float32),
      scratch_shapes=[pltpu.VMEM((1, D), jnp.float32)],
  )(P, degT, x, Wm, Ws, b.reshape(1, D),
    fc1_w, fc1_b.reshape(1, -1), fc2_w, fc2_b.reshape(1, D))


def kernel(x, edge_index, W_msg0, W_self0, b0, W_msg1, W_self1, b1,
           fc1_w, fc1_b, fc2_w, fc2_b):
  src = edge_index[0].astype(jnp.int32).reshape(NW, CH, K)
  dst = edge_index[1].astype(jnp.int32).reshape(NW, CH, K)

  x_pad = jnp.pad(x, ((0, NP - N), (0, 0)))

  P1, deg = _sc_segsum_deg(x_pad, src, dst)
  degT = jnp.transpose(deg)
  h1 = _dense_layer(P1.reshape(2, NP, D), degT, x_pad, W_msg0, W_self0, b0)

  P2 = _sc_segsum(h1, src, dst)
  return _dense2_head(P2.reshape(2, NP, D), degT, h1, W_msg1, W_self1, b1,
                      fc1_w, fc1_b, fc2_w, fc2_b)


# speedup vs baseline: 12.1501x; 1.1157x over previous
import functools

import jax
import jax.numpy as jnp
from jax import lax
from jax.experimental import pallas as pl
from jax.experimental.pallas import tpu as pltpu
from jax.experimental.pallas import tpu_sc as plsc

N = 10000
E = 320000
D = 128

NC = 2
NS = 16
NW = NC * NS
EW = E // NW
K = 50
CH = EW // K
IB = 100
NBLK = CH // IB
NP = 10240
RPT = NP // NS
RZC = 40
KF = K // 16
KR = K - 16 * KF


def _make_sc_segsum(with_deg):
  mesh = plsc.VectorSubcoreMesh(core_axis_name="c", subcore_axis_name="s")

  out_type = jax.ShapeDtypeStruct((2 * NP, D), jnp.float32)
  scratch = [
      pltpu.VMEM((IB, K), jnp.int32),
      pltpu.VMEM((IB, K), jnp.int32),
      pltpu.VMEM((K, D), jnp.float32),
      pltpu.VMEM((K, D), jnp.float32),
      pltpu.VMEM((K, D), jnp.float32),
      pltpu.VMEM((K, D), jnp.float32),
      pltpu.VMEM_SHARED((NP, D), jnp.float32),
      pltpu.SemaphoreType.DMA,
      pltpu.SemaphoreType.DMA,
      pltpu.SemaphoreType.DMA,
      pltpu.SemaphoreType.DMA,
      pltpu.SemaphoreType.DMA,
      pltpu.SemaphoreType.DMA,
      pltpu.SemaphoreType.DMA,
      pltpu.SemaphoreType.DMA,
  ]
  if with_deg:
    out_type = [out_type, jax.ShapeDtypeStruct((NW, NP), jnp.float32)]
    scratch = scratch + [
        pltpu.VMEM((NP,), jnp.float32),
    ]

  @functools.partial(
      pl.kernel,
      out_type=out_type,
      mesh=mesh,
      compiler_params=pltpu.CompilerParams(use_tc_tiling_on_sc=False,
                                           needs_layout_passes=False),
      scratch_types=scratch,
  )
  def segsum(tab_hbm, src_hbm, dst_hbm, out_hbm, *rest):
    if with_deg:
      (deg_hbm, src_v, dst_v, rows0, rows1, rows2, rows3, acc_sh,
       sg0, sg1, sg2, sg3, ss0, ss1, ss2, ss3, degloc) = rest
    else:
      (src_v, dst_v, rows0, rows1, rows2, rows3, acc_sh,
       sg0, sg1, sg2, sg3, ss0, ss1, ss2, ss3) = rest
    slots = ((rows0, sg0, ss0), (rows1, sg1, ss1),
             (rows2, sg2, ss2), (rows3, sg3, ss3))
    cid = lax.axis_index("c")
    sid = lax.axis_index("s")
    wid = cid * NS + sid

    zero = jnp.zeros((16,), jnp.float32)

    def zrow(r, carry):
      def zcol(c, carry2):
        rows0[r, pl.ds(c * 16, 16)] = zero
        return carry2
      return lax.fori_loop(0, D // 16, zcol, carry)

    lax.fori_loop(0, RZC, zrow, 0)
    for j in range(RPT // RZC):
      pltpu.sync_copy(rows0.at[pl.ds(0, RZC)],
                      acc_sh.at[pl.ds(sid * RPT + j * RZC, RZC)])

    if with_deg:
      def zdeg(r, carry):
        degloc[pl.ds(r * 16, 16)] = zero
        return carry
      lax.fori_loop(0, NP // 16, zdeg, 0)

    pltpu.sync_copy(src_hbm.at[wid, pl.ds(0, IB)], src_v)
    pltpu.sync_copy(dst_hbm.at[wid, pl.ds(0, IB)], dst_v)

    plsc.subcore_barrier()

    def g_issue(rows, c, sem):
      pltpu.async_copy(tab_hbm.at[src_v.at[c]], rows, sem)

    def g_wait(rows, c, sem):
      pltpu.make_async_copy(tab_hbm.at[src_v.at[c]], rows, sem).wait()

    def s_issue(rows, c, sem):
      pltpu.async_copy(rows, acc_sh.at[dst_v.at[c]], sem, add=True)

    def s_wait(rows, c, sem):
      pltpu.make_async_copy(rows, acc_sh.at[dst_v.at[c]], sem).wait()

    if with_deg:
      ones16 = jnp.ones((16,), jnp.float32)
      rmask = lax.iota(jnp.int32, 16) >= 16 - KR

      def histo(c):
        for g in range(KF):
          idx = dst_v[c, pl.ds(g * 16, 16)]
          plsc.addupdate_scatter(degloc, [idx], ones16)
        if KR:
          idx = dst_v[c, pl.ds(K - 16, 16)]
          plsc.addupdate_scatter(degloc, [idx], ones16, mask=rmask)
    else:
      def histo(c):
        del c

    for b in range(NBLK):
      if b > 0:
        pltpu.sync_copy(src_hbm.at[wid, pl.ds(b * IB, IB)], src_v)
        pltpu.sync_copy(dst_hbm.at[wid, pl.ds(b * IB, IB)], dst_v)

      for t, (rows, sg, ss) in enumerate(slots):
        g_issue(rows, t, sg)

      def quad(q, carry):
        c0 = 4 * q
        for t, (rows, sg, ss) in enumerate(slots):
          g_wait(rows, c0 + t, sg)
          s_issue(rows, c0 + t, ss)
          histo(c0 + t)
        for t, (rows, sg, ss) in enumerate(slots):
          s_wait(rows, c0 + t, ss)
          g_issue(rows, c0 + t + 4, sg)
        return carry

      lax.fori_loop(0, IB // 4 - 1, quad, 0)

      c0 = IB - 4
      for t, (rows, sg, ss) in enumerate(slots):
        g_wait(rows, c0 + t, sg)
        s_issue(rows, c0 + t, ss)
        histo(c0 + t)
      for t, (rows, sg, ss) in enumerate(slots):
        s_wait(rows, c0 + t, ss)

    if with_deg:
      pltpu.sync_copy(degloc, deg_hbm.at[wid])

    plsc.subcore_barrier()

    row = sid * RPT
    pltpu.sync_copy(acc_sh.at[pl.ds(row, RPT)],
                    out_hbm.at[pl.ds(cid * NP + row, RPT)])

  return segsum


_sc_segsum_deg = _make_sc_segsum(True)
_sc_segsum = _make_sc_segsum(False)


_RB = 2048


def _dense_body(p_ref, d_ref, x_ref, wm_ref, ws_ref, b_ref, o_ref):
  a = p_ref[0] + p_ref[1]
  deg = jnp.sum(d_ref[...], axis=1, keepdims=True)
  scale = 1.0 / jnp.maximum(deg, 1.0)
  agg = jnp.dot(a * scale, wm_ref[...], preferred_element_type=jnp.float32)
  self_t = jnp.dot(x_ref[...], ws_ref[...], preferred_element_type=jnp.float32)
  o_ref[...] = jnp.maximum(agg + self_t + b_ref[...], 0.0)


def _dense_layer(P, degT, x, Wm, Ws, b):
  return pl.pallas_call(
      _dense_body,
      grid=(NP // _RB,),
      in_specs=[
          pl.BlockSpec((2, _RB, D), lambda i: (0, i, 0)),
          pl.BlockSpec((_RB, NW), lambda i: (i, 0)),
          pl.BlockSpec((_RB, D), lambda i: (i, 0)),
          pl.BlockSpec((D, D), lambda i: (0, 0)),
          pl.BlockSpec((D, D), lambda i: (0, 0)),
          pl.BlockSpec((1, D), lambda i: (0, 0)),
      ],
      out_specs=pl.BlockSpec((_RB, D), lambda i: (i, 0)),
      out_shape=jax.ShapeDtypeStruct((NP, D), jnp.float32),
  )(P, degT, x, Wm, Ws, b.reshape(1, D))


def _dense2_head_body(p_ref, d_ref, x_ref, wm_ref, ws_ref, b_ref,
                      w1_ref, b1_ref, w2_ref, b2_ref, o_ref, gmax):
  i = pl.program_id(0)
  a = p_ref[0] + p_ref[1]
  deg = jnp.sum(d_ref[...], axis=1, keepdims=True)
  scale = 1.0 / jnp.maximum(deg, 1.0)
  agg = jnp.dot(a * scale, wm_ref[...], preferred_element_type=jnp.float32)
  self_t = jnp.dot(x_ref[...], ws_ref[...], preferred_element_type=jnp.float32)
  h2 = jnp.maximum(agg + self_t + b_ref[...], 0.0)
  rows = i * _RB + lax.broadcasted_iota(jnp.int32, (_RB, 1), 0)
  bm = jnp.max(jnp.where(rows < N, h2, -jnp.inf), axis=0, keepdims=True)

  @pl.when(i == 0)
  def _():
    gmax[...] = bm

  @pl.when(i > 0)
  def _():
    gmax[...] = jnp.maximum(gmax[...], bm)

  @pl.when(i == NP // _RB - 1)
  def _():
    g = gmax[...]
    h1v = jnp.maximum(
        jnp.dot(g, w1_ref[...], preferred_element_type=jnp.float32)
        + b1_ref[...], 0.0)
    o_ref[...] = jnp.dot(h1v, w2_ref[...],
                         preferred_element_type=jnp.float32) + b2_ref[...]


def _dense2_head(P, degT, x, Wm, Ws, b, fc1_w, fc1_b, fc2_w, fc2_b):
  return pl.pallas_call(
      _dense2_head_body,
      grid=(NP // _RB,),
      in_specs=[
          pl.BlockSpec((2, _RB, D), lambda i: (0, i, 0)),
          pl.BlockSpec((_RB, NW), lambda i: (i, 0)),
          pl.BlockSpec((_RB, D), lambda i: (i, 0)),
          pl.BlockSpec((D, D), lambda i: (0, 0)),
          pl.BlockSpec((D, D), lambda i: (0, 0)),
          pl.BlockSpec((1, D), lambda i: (0, 0)),
          pl.BlockSpec((D, D // 2), lambda i: (0, 0)),
          pl.BlockSpec((1, D // 2), lambda i: (0, 0)),
          pl.BlockSpec((D // 2, D), lambda i: (0, 0)),
          pl.BlockSpec((1, D), lambda i: (0, 0)),
      ],
      out_specs=pl.BlockSpec((1, D), lambda i: (0, 0)),
      out_shape=jax.ShapeDtypeStruct((1, D), jnp.float32),
      scratch_shapes=[pltpu.VMEM((1, D), jnp.float32)],
  )(P, degT, x, Wm, Ws, b.reshape(1, D),
    fc1_w, fc1_b.reshape(1, -1), fc2_w, fc2_b.reshape(1, D))


def kernel(x, edge_index, W_msg0, W_self0, b0, W_msg1, W_self1, b1,
           fc1_w, fc1_b, fc2_w, fc2_b):
  src = edge_index[0].astype(jnp.int32).reshape(NW, CH, K)
  dst = edge_index[1].astype(jnp.int32).reshape(NW, CH, K)

  x_pad = jnp.pad(x, ((0, NP - N), (0, 0)))

  P1, deg = _sc_segsum_deg(x_pad, src, dst)
  degT = jnp.transpose(deg)
  h1 = _dense_layer(P1.reshape(2, NP, D), degT, x_pad, W_msg0, W_self0, b0)

  P2 = _sc_segsum(h1, src, dst)
  return _dense2_head(P2.reshape(2, NP, D), degT, h1, W_msg1, W_self1, b1,
                      fc1_w, fc1_b, fc2_w, fc2_b)


# speedup vs baseline: 12.5453x; 1.0325x over previous
import functools

import jax
import jax.numpy as jnp
from jax import lax
from jax.experimental import pallas as pl
from jax.experimental.pallas import tpu as pltpu
from jax.experimental.pallas import tpu_sc as plsc

N = 10000
E = 320000
D = 128

NC = 2
NS = 16
NW = NC * NS
EW = E // NW
K = 50
CH = EW // K
IB = 100
NBLK = CH // IB
NP = 10240
RPT = NP // NS
RZC = 40
KF = K // 16
KR = K - 16 * KF


def _make_sc_segsum(with_deg):
  mesh = plsc.VectorSubcoreMesh(core_axis_name="c", subcore_axis_name="s")

  out_type = jax.ShapeDtypeStruct((2 * NP, D), jnp.float32)
  scratch = [
      pltpu.VMEM((IB, K), jnp.int32),
      pltpu.VMEM((IB, K), jnp.int32),
      pltpu.VMEM((K, D), jnp.float32),
      pltpu.VMEM((K, D), jnp.float32),
      pltpu.VMEM((K, D), jnp.float32),
      pltpu.VMEM((K, D), jnp.float32),
      pltpu.VMEM_SHARED((NP, D), jnp.float32),
      pltpu.SemaphoreType.DMA,
      pltpu.SemaphoreType.DMA,
      pltpu.SemaphoreType.DMA,
      pltpu.SemaphoreType.DMA,
      pltpu.SemaphoreType.DMA,
      pltpu.SemaphoreType.DMA,
      pltpu.SemaphoreType.DMA,
      pltpu.SemaphoreType.DMA,
  ]
  if with_deg:
    out_type = [out_type, jax.ShapeDtypeStruct((NW, NP), jnp.float32)]
    scratch = scratch + [
        pltpu.VMEM((NP,), jnp.float32),
    ]

  @functools.partial(
      pl.kernel,
      out_type=out_type,
      mesh=mesh,
      compiler_params=pltpu.CompilerParams(use_tc_tiling_on_sc=False,
                                           needs_layout_passes=False),
      scratch_types=scratch,
  )
  def segsum(tab_hbm, edges_hbm, out_hbm, *rest):
    if with_deg:
      (deg_hbm, src_v, dst_v, rows0, rows1, rows2, rows3, acc_sh,
       sg0, sg1, sg2, sg3, ss0, ss1, ss2, ss3, degloc) = rest
    else:
      (src_v, dst_v, rows0, rows1, rows2, rows3, acc_sh,
       sg0, sg1, sg2, sg3, ss0, ss1, ss2, ss3) = rest
    slots = ((rows0, sg0, ss0), (rows1, sg1, ss1),
             (rows2, sg2, ss2), (rows3, sg3, ss3))
    cid = lax.axis_index("c")
    sid = lax.axis_index("s")
    wid = cid * NS + sid

    zero = jnp.zeros((16,), jnp.float32)

    def zrow(r, carry):
      def zcol(c, carry2):
        rows0[r, pl.ds(c * 16, 16)] = zero
        return carry2
      return lax.fori_loop(0, D // 16, zcol, carry)

    lax.fori_loop(0, RZC, zrow, 0)
    for j in range(RPT // RZC):
      pltpu.sync_copy(rows0.at[pl.ds(0, RZC)],
                      acc_sh.at[pl.ds(sid * RPT + j * RZC, RZC)])

    if with_deg:
      def zdeg(r, carry):
        degloc[pl.ds(r * 16, 16)] = zero
        return carry
      lax.fori_loop(0, NP // 16, zdeg, 0)

    pltpu.sync_copy(edges_hbm.at[0, wid, pl.ds(0, IB)], src_v)
    pltpu.sync_copy(edges_hbm.at[1, wid, pl.ds(0, IB)], dst_v)

    plsc.subcore_barrier()

    def g_issue(rows, c, sem):
      pltpu.async_copy(tab_hbm.at[src_v.at[c]], rows, sem)

    def g_wait(rows, c, sem):
      pltpu.make_async_copy(tab_hbm.at[src_v.at[c]], rows, sem).wait()

    def s_issue(rows, c, sem):
      pltpu.async_copy(rows, acc_sh.at[dst_v.at[c]], sem, add=True)

    def s_wait(rows, c, sem):
      pltpu.make_async_copy(rows, acc_sh.at[dst_v.at[c]], sem).wait()

    if with_deg:
      ones16 = jnp.ones((16,), jnp.float32)
      rmask = lax.iota(jnp.int32, 16) >= 16 - KR

      def histo(c):
        for g in range(KF):
          idx = dst_v[c, pl.ds(g * 16, 16)]
          plsc.addupdate_scatter(degloc, [idx], ones16)
        if KR:
          idx = dst_v[c, pl.ds(K - 16, 16)]
          plsc.addupdate_scatter(degloc, [idx], ones16, mask=rmask)
    else:
      def histo(c):
        del c

    for b in range(NBLK):
      if b > 0:
        pltpu.sync_copy(edges_hbm.at[0, wid, pl.ds(b * IB, IB)], src_v)
        pltpu.sync_copy(edges_hbm.at[1, wid, pl.ds(b * IB, IB)], dst_v)

      for t, (rows, sg, ss) in enumerate(slots):
        g_issue(rows, t, sg)

      def quad(q, carry):
        c0 = 4 * q
        for t, (rows, sg, ss) in enumerate(slots):
          g_wait(rows, c0 + t, sg)
          s_issue(rows, c0 + t, ss)
          histo(c0 + t)
        for t, (rows, sg, ss) in enumerate(slots):
          s_wait(rows, c0 + t, ss)
          g_issue(rows, c0 + t + 4, sg)
        return carry

      lax.fori_loop(0, IB // 4 - 1, quad, 0)

      c0 = IB - 4
      for t, (rows, sg, ss) in enumerate(slots):
        g_wait(rows, c0 + t, sg)
        s_issue(rows, c0 + t, ss)
        histo(c0 + t)
      for t, (rows, sg, ss) in enumerate(slots):
        s_wait(rows, c0 + t, ss)

    if with_deg:
      pltpu.sync_copy(degloc, deg_hbm.at[wid])

    plsc.subcore_barrier()

    row = sid * RPT
    pltpu.sync_copy(acc_sh.at[pl.ds(row, RPT)],
                    out_hbm.at[pl.ds(cid * NP + row, RPT)])

  return segsum


_sc_segsum_deg = _make_sc_segsum(True)
_sc_segsum = _make_sc_segsum(False)


_RB = 2048


def _dense_body(p_ref, d_ref, x_ref, wm_ref, ws_ref, b_ref, o_ref):
  a = p_ref[0] + p_ref[1]
  deg = jnp.sum(d_ref[...], axis=1, keepdims=True)
  scale = 1.0 / jnp.maximum(deg, 1.0)
  agg = jnp.dot(a * scale, wm_ref[...], preferred_element_type=jnp.float32)
  self_t = jnp.dot(x_ref[...], ws_ref[...], preferred_element_type=jnp.float32)
  o_ref[...] = jnp.maximum(agg + self_t + b_ref[...], 0.0)


def _dense_layer(P, degT, x, Wm, Ws, b):
  return pl.pallas_call(
      _dense_body,
      grid=(NP // _RB,),
      in_specs=[
          pl.BlockSpec((2, _RB, D), lambda i: (0, i, 0)),
          pl.BlockSpec((_RB, NW), lambda i: (i, 0)),
          pl.BlockSpec((_RB, D), lambda i: (i, 0)),
          pl.BlockSpec((D, D), lambda i: (0, 0)),
          pl.BlockSpec((D, D), lambda i: (0, 0)),
          pl.BlockSpec((1, D), lambda i: (0, 0)),
      ],
      out_specs=pl.BlockSpec((_RB, D), lambda i: (i, 0)),
      out_shape=jax.ShapeDtypeStruct((NP, D), jnp.float32),
  )(P, degT, x, Wm, Ws, b.reshape(1, D))


def _dense2_head_body(p_ref, d_ref, x_ref, wm_ref, ws_ref, b_ref,
                      w1_ref, b1_ref, w2_ref, b2_ref, o_ref, gmax):
  i = pl.program_id(0)
  a = p_ref[0] + p_ref[1]
  deg = jnp.sum(d_ref[...], axis=1, keepdims=True)
  scale = 1.0 / jnp.maximum(deg, 1.0)
  agg = jnp.dot(a * scale, wm_ref[...], preferred_element_type=jnp.float32)
  self_t = jnp.dot(x_ref[...], ws_ref[...], preferred_element_type=jnp.float32)
  h2 = jnp.maximum(agg + self_t + b_ref[...], 0.0)
  rows = i * _RB + lax.broadcasted_iota(jnp.int32, (_RB, 1), 0)
  bm = jnp.max(jnp.where(rows < N, h2, -jnp.inf), axis=0, keepdims=True)

  @pl.when(i == 0)
  def _():
    gmax[...] = bm

  @pl.when(i > 0)
  def _():
    gmax[...] = jnp.maximum(gmax[...], bm)

  @pl.when(i == NP // _RB - 1)
  def _():
    g = gmax[...]
    h1v = jnp.maximum(
        jnp.dot(g, w1_ref[...], preferred_element_type=jnp.float32)
        + b1_ref[...], 0.0)
    o_ref[...] = jnp.dot(h1v, w2_ref[...],
                         preferred_element_type=jnp.float32) + b2_ref[...]


def _dense2_head(P, degT, x, Wm, Ws, b, fc1_w, fc1_b, fc2_w, fc2_b):
  return pl.pallas_call(
      _dense2_head_body,
      grid=(NP // _RB,),
      in_specs=[
          pl.BlockSpec((2, _RB, D), lambda i: (0, i, 0)),
          pl.BlockSpec((_RB, NW), lambda i: (i, 0)),
          pl.BlockSpec((_RB, D), lambda i: (i, 0)),
          pl.BlockSpec((D, D), lambda i: (0, 0)),
          pl.BlockSpec((D, D), lambda i: (0, 0)),
          pl.BlockSpec((1, D), lambda i: (0, 0)),
          pl.BlockSpec((D, D // 2), lambda i: (0, 0)),
          pl.BlockSpec((1, D // 2), lambda i: (0, 0)),
          pl.BlockSpec((D // 2, D), lambda i: (0, 0)),
          pl.BlockSpec((1, D), lambda i: (0, 0)),
      ],
      out_specs=pl.BlockSpec((1, D), lambda i: (0, 0)),
      out_shape=jax.ShapeDtypeStruct((1, D), jnp.float32),
      scratch_shapes=[pltpu.VMEM((1, D), jnp.float32)],
  )(P, degT, x, Wm, Ws, b.reshape(1, D),
    fc1_w, fc1_b.reshape(1, -1), fc2_w, fc2_b.reshape(1, D))


def kernel(x, edge_index, W_msg0, W_self0, b0, W_msg1, W_self1, b1,
           fc1_w, fc1_b, fc2_w, fc2_b):
  edges = edge_index.astype(jnp.int32).reshape(2, NW, CH, K)

  x_pad = jnp.pad(x, ((0, NP - N), (0, 0)))

  P1, deg = _sc_segsum_deg(x_pad, edges)
  degT = jnp.transpose(deg)
  h1 = _dense_layer(P1.reshape(2, NP, D), degT, x_pad, W_msg0, W_self0, b0)

  P2 = _sc_segsum(h1, edges)
  return _dense2_head(P2.reshape(2, NP, D), degT, h1, W_msg1, W_self1, b1,
                      fc1_w, fc1_b, fc2_w, fc2_b)
